# baseline (speedup 1.0000x reference)
import functools

import jax
import jax.numpy as jnp
from jax import lax
from jax.experimental import pallas as pl
from jax.experimental.pallas import tpu as pltpu
from jax.experimental.pallas import tpu_sc as plsc

N_CH = [32, 64, 128, 256]
K = 16
NS_LVL = [8192, 2048, 512, 128]
B = 2

NC = 2
NSUB = 16
NW = NC * NSUB

F32 = jnp.float32
HI = jax.lax.Precision.HIGHEST


def _lrelu(x):
    return jnp.maximum(x, 0.1 * x)


def _dot(a, b):
    return jnp.dot(a, b, precision=HI, preferred_element_type=F32)


def _stem_body(x_ref, w1t, b1, w2t, b2, mw1t, mb1, mw2t, mb2, cwft, cwrt,
               f_ref, u_ref):
    x = x_ref[...]
    f = _lrelu(_dot(_lrelu(_dot(x, w1t[...]) + b1[...]), w2t[...]) + b2[...])
    g = _lrelu(_dot(_lrelu(_dot(f, mw1t[...]) + mb1[...]), mw2t[...]) + mb2[...])
    f_ref[...] = f
    u = _dot(g, cwft[...]) + _dot(x, cwrt[...])
    u_ref[...] = jnp.concatenate([u, jnp.zeros_like(u)], axis=1)


def _stem(x_rows, l0_W1, l0_b1, l0_W2, l0_b2, m_W1, m_b1, m_W2, m_b2,
          c_Wf, c_Wr):
    R = x_rows.shape[0]
    NB = 512
    full = lambda a: pl.BlockSpec(a.shape, lambda i: (0,) * a.ndim)
    args = (l0_W1.T, l0_b1.reshape(1, -1), l0_W2.T, l0_b2.reshape(1, -1),
            m_W1.T, m_b1.reshape(1, -1), m_W2.T, m_b2.reshape(1, -1),
            c_Wf.T, c_Wr.T)
    return pl.pallas_call(
        _stem_body,
        grid=(R // NB,),
        in_specs=[pl.BlockSpec((NB, 3), lambda i: (i, 0))] + [full(a) for a in args],
        out_specs=[pl.BlockSpec((NB, 32), lambda i: (i, 0)),
                   pl.BlockSpec((NB, 128), lambda i: (i, 0))],
        out_shape=[jax.ShapeDtypeStruct((R, 32), F32),
                   jax.ShapeDtypeStruct((R, 128), F32)],
    )(x_rows, *args)


def _prep_body(x_ref, f_ref, mw1t, mb1, mw2t, mb2, cwft, cwrt, u_ref):
    g = _lrelu(_dot(_lrelu(_dot(f_ref[...], mw1t[...]) + mb1[...]),
                    mw2t[...]) + mb2[...])
    u_ref[...] = _dot(g, cwft[...]) + _dot(x_ref[...], cwrt[...])


def _prep(x_rows, f_rows, m_W1, m_b1, m_W2, m_b2, c_Wf, c_Wr):
    R, ci = f_rows.shape
    co = m_W2.shape[0]
    NB = min(512, R)
    full = lambda a: pl.BlockSpec(a.shape, lambda i: (0,) * a.ndim)
    args = (m_W1.T, m_b1.reshape(1, -1), m_W2.T, m_b2.reshape(1, -1),
            c_Wf.T, c_Wr.T)
    return pl.pallas_call(
        _prep_body,
        grid=(R // NB,),
        in_specs=[pl.BlockSpec((NB, 3), lambda i: (i, 0)),
                  pl.BlockSpec((NB, ci), lambda i: (i, 0))]
                 + [full(a) for a in args],
        out_specs=pl.BlockSpec((NB, co), lambda i: (i, 0)),
        out_shape=jax.ShapeDtypeStruct((R, co), F32),
    )(x_rows, f_rows, *args)


def _knn_body(src_ref, dst_ref, cwr_ref, cb_ref, idx_ref, c_ref, *, n, mb):
    src = src_ref[0]
    dst = dst_ref[0]
    q = n // 4
    dstn = jnp.sum(dst * dst, axis=0)[:, None]
    giota = lax.broadcasted_iota(jnp.int32, (mb, q), 1)
    grp = []
    for j in range(4):
        srcq = src[:, j * q:(j + 1) * q]
        dot = lax.dot_general(dst, srcq, (((0,), (0,)), ((), ())),
                              precision=lax.Precision.DEFAULT,
                              preferred_element_type=F32)
        srcn = jnp.sum(srcq * srcq, axis=0, keepdims=True)
        grp.append((-(dstn + srcn - 2.0 * dot), giota + j * q))

    def ce(a, b):
        c = b[0] > a[0]
        hi = (jnp.where(c, b[0], a[0]), jnp.where(c, b[1], a[1]))
        lo = (jnp.where(c, a[0], b[0]), jnp.where(c, a[1], b[1]))
        return hi, lo

    grp[0], grp[1] = ce(grp[0], grp[1])
    grp[2], grp[3] = ce(grp[2], grp[3])
    grp[0], grp[2] = ce(grp[0], grp[2])
    grp[1], grp[3] = ce(grp[1], grp[3])
    grp[1], grp[2] = ce(grp[1], grp[2])
    (v1, i1), (v2, i2), (v3, i3), (v4, i4) = grp

    cols = []
    for _ in range(K):
        m = jnp.max(v1, axis=1, keepdims=True)
        idxt = jnp.min(jnp.where(v1 == m, i1, n), axis=1, keepdims=True)
        cols.append(idxt)
        upd = i1 == idxt
        v1 = jnp.where(upd, v2, v1)
        i1 = jnp.where(upd, i2, i1)
        v2 = jnp.where(upd, v3, v2)
        i2 = jnp.where(upd, i3, i2)
        v3 = jnp.where(upd, v4, v3)
        i3 = jnp.where(upd, i4, i3)
        v4 = jnp.where(upd, -jnp.inf, v4)
    b = pl.program_id(0)
    idx_ref[0] = jnp.concatenate(cols, axis=1) + b * n
    cq = lax.dot_general(dst, cwr_ref[...], (((0,), (1,)), ((), ())),
                         precision=HI, preferred_element_type=F32)
    c_ref[0] = cb_ref[...] - cq


def _knn(src, dst, c_Wr, c_b, mb):
    _, _, n = src.shape
    m = dst.shape[2]
    co = c_Wr.shape[0]
    cb = c_b.reshape(1, -1)
    return pl.pallas_call(
        functools.partial(_knn_body, n=n, mb=mb),
        grid=(B, m // mb),
        in_specs=[pl.BlockSpec((1, 3, n), lambda b, j: (b, 0, 0)),
                  pl.BlockSpec((1, 3, mb), lambda b, j: (b, 0, j)),
                  pl.BlockSpec(c_Wr.shape, lambda b, j: (0, 0)),
                  pl.BlockSpec(cb.shape, lambda b, j: (0, 0))],
        out_specs=[pl.BlockSpec((1, mb, K), lambda b, j: (b, j, 0)),
                   pl.BlockSpec((1, mb, co), lambda b, j: (b, j, 0))],
        out_shape=[jax.ShapeDtypeStruct((B, m, K), jnp.int32),
                   jax.ShapeDtypeStruct((B, m, co), F32)],
        compiler_params=pltpu.CompilerParams(
            dimension_semantics=("parallel", "parallel")),
    )(src, dst, c_Wr, cb)


def _gathermax(u_rows, idx3d, c_rows):
    T, co = c_rows.shape
    cw = u_rows.shape[1]
    t_per_w = T // NW
    G = min(8, t_per_w)
    n_groups = t_per_w // G
    nj = co // 16

    mesh = plsc.VectorSubcoreMesh(core_axis_name="c", subcore_axis_name="s",
                                  num_cores=NC, num_subcores=NSUB)

    @functools.partial(
        pl.kernel,
        out_type=jax.ShapeDtypeStruct((T, co), F32),
        mesh=mesh,
        scratch_types=[
            pltpu.VMEM((n_groups, G * K), jnp.int32),
            pltpu.VMEM((t_per_w, co), F32),
            pltpu.VMEM((G * K, cw), F32),
            pltpu.VMEM((t_per_w, co), F32),
            pltpu.SemaphoreType.DMA,
        ],
    )
    def sck(u_hbm, idx_hbm, c_hbm, out_hbm, idx_v, c_v, rows_v, out_v, sem):
        wid = lax.axis_index("s") * NC + lax.axis_index("c")
        base = wid * t_per_w
        pltpu.sync_copy(idx_hbm.at[wid], idx_v)
        pltpu.sync_copy(c_hbm.at[pl.ds(base, t_per_w)], c_v)

        def group_body(g, carry):
            pltpu.async_copy(u_hbm.at[idx_v.at[g]], rows_v, sem).wait()

            def tgt_body(t, carry2):
                tg = g * G + t
                for j in range(nj):
                    cj = c_v[tg, pl.ds(j * 16, 16)]
                    acc = jnp.full((16,), -jnp.inf, F32)
                    for k in range(K):
                        z = rows_v[t * K + k, pl.ds(j * 16, 16)] + cj
                        acc = jnp.maximum(acc, jnp.maximum(z, 0.1 * z))
                    out_v[tg, pl.ds(j * 16, 16)] = acc
                return carry2

            return lax.fori_loop(0, G, tgt_body, carry)

        lax.fori_loop(0, n_groups, group_body, 0)
        pltpu.sync_copy(out_v, out_hbm.at[pl.ds(base, t_per_w)])

    return sck(u_rows, idx3d, c_rows)


def kernel(xyz0, xyz1, xyz2, xyz3, l0_W1, l0_b1, l0_W2, l0_b2,
           m0_W1, m0_b1, m0_W2, m0_b2, c0_W, c0_b,
           m1_W1, m1_b1, m1_W2, m1_b2, c1_W, c1_b,
           m2_W1, m2_b1, m2_W2, m2_b2, c2_W, c2_b):
    xyzs = [xyz0, xyz1, xyz2, xyz3]
    xrows = [jnp.transpose(x, (0, 2, 1)).reshape(-1, 3) for x in xyzs]
    mlps = [(m0_W1, m0_b1, m0_W2, m0_b2), (m1_W1, m1_b1, m1_W2, m1_b2),
            (m2_W1, m2_b1, m2_W2, m2_b2)]
    convs = [(c0_W, c0_b), (c1_W, c1_b), (c2_W, c2_b)]
    mbs = [512, 256, 128]

    knns = []
    for i in range(3):
        co = N_CH[i + 1]
        c_W, c_b = convs[i]
        knns.append(_knn(xyzs[i], xyzs[i + 1], c_W[:, co:], c_b, mbs[i]))

    feats_rows = []
    f0, u = _stem(xrows[0], l0_W1, l0_b1, l0_W2, l0_b2,
                  m0_W1, m0_b1, m0_W2, m0_b2,
                  c0_W[:, :N_CH[1]], c0_W[:, N_CH[1]:])
    feats_rows.append(f0)
    for i in range(3):
        co = N_CH[i + 1]
        c_W, c_b = convs[i]
        c_Wf, c_Wr = c_W[:, :co], c_W[:, co:]
        if i > 0:
            u = _prep(xrows[i], feats_rows[i], *mlps[i], c_Wf, c_Wr)
        m = NS_LVL[i + 1]
        idx, c = knns[i]
        t_per_w = (B * m) // NW
        g = min(8, t_per_w)
        idx3d = idx.reshape(NW, t_per_w // g, g * K)
        f = _gathermax(u, idx3d, c.reshape(B * m, co))
        feats_rows.append(f)

    out = []
    for i, fr in enumerate(feats_rows):
        out.append(jnp.transpose(fr.reshape(B, NS_LVL[i], N_CH[i]), (0, 2, 1)))
    return tuple(out)


# speedup vs baseline: 1.1920x; 1.1920x over previous
import functools

import jax
import jax.numpy as jnp
from jax import lax
from jax.experimental import pallas as pl
from jax.experimental.pallas import tpu as pltpu
from jax.experimental.pallas import tpu_sc as plsc

N_CH = [32, 64, 128, 256]
K = 16
NS_LVL = [8192, 2048, 512, 128]
B = 2

NC = 2
NSUB = 16
NW = NC * NSUB

F32 = jnp.float32
HI = jax.lax.Precision.HIGHEST


def _lrelu(x):
    return jnp.maximum(x, 0.1 * x)


def _dot(a, b):
    return jnp.dot(a, b, precision=HI, preferred_element_type=F32)


def _stem_body(x_ref, w1t, b1, w2t, b2, mw1t, mb1, mw2t, mb2, cwft, cwrt,
               f_ref, u_ref):
    x = x_ref[...]
    f = _lrelu(_dot(_lrelu(_dot(x, w1t[...]) + b1[...]), w2t[...]) + b2[...])
    g = _lrelu(_dot(_lrelu(_dot(f, mw1t[...]) + mb1[...]), mw2t[...]) + mb2[...])
    f_ref[...] = f
    u = _dot(g, cwft[...]) + _dot(x, cwrt[...])
    u_ref[...] = jnp.concatenate([u, jnp.zeros_like(u)], axis=1)


def _stem(x_rows, l0_W1, l0_b1, l0_W2, l0_b2, m_W1, m_b1, m_W2, m_b2,
          c_Wf, c_Wr):
    R = x_rows.shape[0]
    NB = 512
    full = lambda a: pl.BlockSpec(a.shape, lambda i: (0,) * a.ndim)
    args = (l0_W1.T, l0_b1.reshape(1, -1), l0_W2.T, l0_b2.reshape(1, -1),
            m_W1.T, m_b1.reshape(1, -1), m_W2.T, m_b2.reshape(1, -1),
            c_Wf.T, c_Wr.T)
    return pl.pallas_call(
        _stem_body,
        grid=(R // NB,),
        in_specs=[pl.BlockSpec((NB, 3), lambda i: (i, 0))] + [full(a) for a in args],
        out_specs=[pl.BlockSpec((NB, 32), lambda i: (i, 0)),
                   pl.BlockSpec((NB, 128), lambda i: (i, 0))],
        out_shape=[jax.ShapeDtypeStruct((R, 32), F32),
                   jax.ShapeDtypeStruct((R, 128), F32)],
    )(x_rows, *args)


def _prep_body(x_ref, f_ref, mw1t, mb1, mw2t, mb2, cwft, cwrt, u_ref):
    g = _lrelu(_dot(_lrelu(_dot(f_ref[...], mw1t[...]) + mb1[...]),
                    mw2t[...]) + mb2[...])
    u_ref[...] = _dot(g, cwft[...]) + _dot(x_ref[...], cwrt[...])


def _prep(x_rows, f_rows, m_W1, m_b1, m_W2, m_b2, c_Wf, c_Wr):
    R, ci = f_rows.shape
    co = m_W2.shape[0]
    NB = min(512, R)
    full = lambda a: pl.BlockSpec(a.shape, lambda i: (0,) * a.ndim)
    args = (m_W1.T, m_b1.reshape(1, -1), m_W2.T, m_b2.reshape(1, -1),
            c_Wf.T, c_Wr.T)
    return pl.pallas_call(
        _prep_body,
        grid=(R // NB,),
        in_specs=[pl.BlockSpec((NB, 3), lambda i: (i, 0)),
                  pl.BlockSpec((NB, ci), lambda i: (i, 0))]
                 + [full(a) for a in args],
        out_specs=pl.BlockSpec((NB, co), lambda i: (i, 0)),
        out_shape=jax.ShapeDtypeStruct((R, co), F32),
    )(x_rows, f_rows, *args)


def _knn_body(src_ref, dst_ref, cwr_ref, cb_ref, idx_ref, c_ref, *, n, mb):
    src = src_ref[0]
    dst = dst_ref[0]
    q = n // 4
    dstn = jnp.sum(dst * dst, axis=0)[:, None]
    giota = lax.broadcasted_iota(jnp.int32, (mb, q), 1).astype(F32)
    grp = []
    for j in range(4):
        srcq = src[:, j * q:(j + 1) * q]
        dot = lax.dot_general(dst, srcq, (((0,), (0,)), ((), ())),
                              precision=lax.Precision.DEFAULT,
                              preferred_element_type=F32)
        srcn = jnp.sum(srcq * srcq, axis=0, keepdims=True)
        grp.append((-(dstn + srcn - 2.0 * dot), giota + j * q))

    def ce(a, b):
        c = b[0] > a[0]
        hi = (jnp.where(c, b[0], a[0]), jnp.where(c, b[1], a[1]))
        lo = (jnp.where(c, a[0], b[0]), jnp.where(c, a[1], b[1]))
        return hi, lo

    grp[0], grp[1] = ce(grp[0], grp[1])
    grp[2], grp[3] = ce(grp[2], grp[3])
    grp[0], grp[2] = ce(grp[0], grp[2])
    grp[1], grp[3] = ce(grp[1], grp[3])
    grp[1], grp[2] = ce(grp[1], grp[2])
    (v1, i1), (v2, i2), (v3, i3), (v4, i4) = grp

    cols = []
    for _ in range(K):
        m = jnp.max(v1, axis=1, keepdims=True)
        idxt = jnp.min(jnp.where(v1 == m, i1, float(n)), axis=1,
                       keepdims=True)
        cols.append(idxt.astype(jnp.int32))
        upd = i1 == idxt
        v1 = jnp.where(upd, v2, v1)
        i1 = jnp.where(upd, i2, i1)
        v2 = jnp.where(upd, v3, v2)
        i2 = jnp.where(upd, i3, i2)
        v3 = jnp.where(upd, v4, v3)
        i3 = jnp.where(upd, i4, i3)
        v4 = jnp.where(upd, -jnp.inf, v4)
    b = pl.program_id(0)
    idx_ref[0] = jnp.concatenate(cols, axis=1) + b * n
    cq = lax.dot_general(dst, cwr_ref[...], (((0,), (1,)), ((), ())),
                         precision=HI, preferred_element_type=F32)
    c_ref[0] = cb_ref[...] - cq


def _knn(src, dst, c_Wr, c_b, mb):
    _, _, n = src.shape
    m = dst.shape[2]
    co = c_Wr.shape[0]
    cb = c_b.reshape(1, -1)
    return pl.pallas_call(
        functools.partial(_knn_body, n=n, mb=mb),
        grid=(B, m // mb),
        in_specs=[pl.BlockSpec((1, 3, n), lambda b, j: (b, 0, 0)),
                  pl.BlockSpec((1, 3, mb), lambda b, j: (b, 0, j)),
                  pl.BlockSpec(c_Wr.shape, lambda b, j: (0, 0)),
                  pl.BlockSpec(cb.shape, lambda b, j: (0, 0))],
        out_specs=[pl.BlockSpec((1, mb, K), lambda b, j: (b, j, 0)),
                   pl.BlockSpec((1, mb, co), lambda b, j: (b, j, 0))],
        out_shape=[jax.ShapeDtypeStruct((B, m, K), jnp.int32),
                   jax.ShapeDtypeStruct((B, m, co), F32)],
        compiler_params=pltpu.CompilerParams(
            dimension_semantics=("parallel", "parallel")),
    )(src, dst, c_Wr, cb)


def _gathermax(u_rows, idx3d, c_rows):
    T, co = c_rows.shape
    cw = u_rows.shape[1]
    t_per_w = T // NW
    G = min(8, t_per_w)
    n_groups = t_per_w // G
    nj = co // 16

    mesh = plsc.VectorSubcoreMesh(core_axis_name="c", subcore_axis_name="s",
                                  num_cores=NC, num_subcores=NSUB)

    @functools.partial(
        pl.kernel,
        out_type=jax.ShapeDtypeStruct((T, co), F32),
        mesh=mesh,
        scratch_types=[
            pltpu.VMEM((n_groups, G * K), jnp.int32),
            pltpu.VMEM((t_per_w, co), F32),
            pltpu.VMEM((G * K, cw), F32),
            pltpu.VMEM((t_per_w, co), F32),
            pltpu.SemaphoreType.DMA,
        ],
    )
    def sck(u_hbm, idx_hbm, c_hbm, out_hbm, idx_v, c_v, rows_v, out_v, sem):
        wid = lax.axis_index("s") * NC + lax.axis_index("c")
        base = wid * t_per_w
        pltpu.sync_copy(idx_hbm.at[wid], idx_v)
        pltpu.sync_copy(c_hbm.at[pl.ds(base, t_per_w)], c_v)

        def group_body(g, carry):
            pltpu.async_copy(u_hbm.at[idx_v.at[g]], rows_v, sem).wait()

            def tgt_body(t, carry2):
                tg = g * G + t
                for j in range(nj):
                    cj = c_v[tg, pl.ds(j * 16, 16)]
                    acc = jnp.full((16,), -jnp.inf, F32)
                    for k in range(K):
                        z = rows_v[t * K + k, pl.ds(j * 16, 16)] + cj
                        acc = jnp.maximum(acc, jnp.maximum(z, 0.1 * z))
                    out_v[tg, pl.ds(j * 16, 16)] = acc
                return carry2

            return lax.fori_loop(0, G, tgt_body, carry)

        lax.fori_loop(0, n_groups, group_body, 0)
        pltpu.sync_copy(out_v, out_hbm.at[pl.ds(base, t_per_w)])

    return sck(u_rows, idx3d, c_rows)


def kernel(xyz0, xyz1, xyz2, xyz3, l0_W1, l0_b1, l0_W2, l0_b2,
           m0_W1, m0_b1, m0_W2, m0_b2, c0_W, c0_b,
           m1_W1, m1_b1, m1_W2, m1_b2, c1_W, c1_b,
           m2_W1, m2_b1, m2_W2, m2_b2, c2_W, c2_b):
    xyzs = [xyz0, xyz1, xyz2, xyz3]
    xrows = [jnp.transpose(x, (0, 2, 1)).reshape(-1, 3) for x in xyzs]
    mlps = [(m0_W1, m0_b1, m0_W2, m0_b2), (m1_W1, m1_b1, m1_W2, m1_b2),
            (m2_W1, m2_b1, m2_W2, m2_b2)]
    convs = [(c0_W, c0_b), (c1_W, c1_b), (c2_W, c2_b)]
    mbs = [256, 256, 128]

    knns = []
    for i in range(3):
        co = N_CH[i + 1]
        c_W, c_b = convs[i]
        knns.append(_knn(xyzs[i], xyzs[i + 1], c_W[:, co:], c_b, mbs[i]))

    feats_rows = []
    f0, u = _stem(xrows[0], l0_W1, l0_b1, l0_W2, l0_b2,
                  m0_W1, m0_b1, m0_W2, m0_b2,
                  c0_W[:, :N_CH[1]], c0_W[:, N_CH[1]:])
    feats_rows.append(f0)
    for i in range(3):
        co = N_CH[i + 1]
        c_W, c_b = convs[i]
        c_Wf, c_Wr = c_W[:, :co], c_W[:, co:]
        if i > 0:
            u = _prep(xrows[i], feats_rows[i], *mlps[i], c_Wf, c_Wr)
        m = NS_LVL[i + 1]
        idx, c = knns[i]
        t_per_w = (B * m) // NW
        g = min(8, t_per_w)
        idx3d = idx.reshape(NW, t_per_w // g, g * K)
        f = _gathermax(u, idx3d, c.reshape(B * m, co))
        feats_rows.append(f)

    out = []
    for i, fr in enumerate(feats_rows):
        out.append(jnp.transpose(fr.reshape(B, NS_LVL[i], N_CH[i]), (0, 2, 1)))
    return tuple(out)
